# Initial kernel scaffold; baseline (speedup 1.0000x reference)
import functools

import jax
import jax.numpy as jnp
from jax import lax
from jax.experimental import pallas as pl
from jax.experimental.pallas import tpu as pltpu
from jax.experimental.pallas import tpu_sc as plsc

N = 10000
C = 128
H = 128
T = 2
E = 320000

NC = 2
NS = 16
NP = 10240

_ROWS_PER_TILE = NP // NS
_ECHUNK = 128
_NCHUNKS = E // _ECHUNK

_mesh = plsc.VectorSubcoreMesh(core_axis_name="c", subcore_axis_name="s")


@functools.partial(
    pl.kernel,
    out_type=jax.ShapeDtypeStruct((NC * NP,), jnp.float32),
    mesh=_mesh,
    scratch_types=[
        pltpu.VMEM((_ECHUNK,), jnp.int32),
        pltpu.VMEM((_ECHUNK,), jnp.float32),
        pltpu.VMEM_SHARED((NP,), jnp.float32),
    ],
)
def _deg_kernel(col_hbm, ones_hbm, zeros_hbm, out_hbm, col_v, ones_v, deg_sh):
    c = lax.axis_index("c")
    s = lax.axis_index("s")
    wid = c * NS + s
    r0 = s * _ROWS_PER_TILE
    pltpu.sync_copy(zeros_hbm, deg_sh.at[pl.ds(r0, _ROWS_PER_TILE)])
    pltpu.sync_copy(ones_hbm, ones_v)
    plsc.subcore_barrier()

    nch = jnp.where(wid < _NCHUNKS % (NC * NS),
                    _NCHUNKS // (NC * NS) + 1, _NCHUNKS // (NC * NS))

    def body(k, carry):
        e0 = pl.multiple_of((wid + k * NC * NS) * _ECHUNK, _ECHUNK)
        pltpu.sync_copy(col_hbm.at[pl.ds(e0, _ECHUNK)], col_v)
        pltpu.sync_copy(ones_v, deg_sh.at[col_v], add=True)
        return carry

    lax.fori_loop(0, nch, body, 0)
    plsc.subcore_barrier()
    pltpu.sync_copy(deg_sh.at[pl.ds(r0, _ROWS_PER_TILE)],
                    out_hbm.at[pl.ds(c * NP + r0, _ROWS_PER_TILE)])


_BN_MM = 1000


def _mm_body(x_ref, w_ref, b_ref, degp_ref, hp0_ref, hp1_ref):
    dsum = degp_ref[0, :] + degp_ref[1, :] + 1.0
    dinv = lax.rsqrt(dsum)
    h = jnp.dot(x_ref[...], w_ref[...], preferred_element_type=jnp.float32)
    hp = (h + b_ref[...]) * dinv[:, None]
    hp0_ref[...] = hp[:, :H]
    hp1_ref[...] = hp[:, H:]


def _matmul(x, wcat, bcat, degp):
    return pl.pallas_call(
        _mm_body,
        grid=(N // _BN_MM,),
        in_specs=[
            pl.BlockSpec((_BN_MM, C), lambda i: (i, 0)),
            pl.BlockSpec((C, 2 * H), lambda i: (0, 0)),
            pl.BlockSpec((1, 2 * H), lambda i: (0, 0)),
            pl.BlockSpec((NC, _BN_MM), lambda i: (0, i)),
        ],
        out_specs=[
            pl.BlockSpec((_BN_MM, H), lambda i: (i, 0)),
            pl.BlockSpec((_BN_MM, H), lambda i: (i, 0)),
        ],
        out_shape=[
            jax.ShapeDtypeStruct((N, H), jnp.float32),
            jax.ShapeDtypeStruct((N, H), jnp.float32),
        ],
    )(x, wcat, bcat, degp)


@functools.partial(
    pl.kernel,
    out_type=jax.ShapeDtypeStruct((NC * NP, H), jnp.float32),
    mesh=_mesh,
    scratch_types=[
        pltpu.VMEM((_ECHUNK,), jnp.int32),
        pltpu.VMEM((_ECHUNK,), jnp.int32),
        pltpu.VMEM((_ECHUNK, H), jnp.float32),
        pltpu.VMEM_SHARED((NP, H), jnp.float32),
        pltpu.SemaphoreType.DMA,
    ],
)
def _agg_kernel(row_hbm, col_hbm, hp0_hbm, hp1_hbm, zeros_hbm, out_hbm,
                row_v, col_v, rows_v, acc_sh, sem):
    c = lax.axis_index("c")
    s = lax.axis_index("s")
    r0 = s * _ROWS_PER_TILE
    pltpu.sync_copy(zeros_hbm, acc_sh.at[pl.ds(r0, _ROWS_PER_TILE)])
    plsc.subcore_barrier()

    nch = jnp.where(s < _NCHUNKS % NS, _NCHUNKS // NS + 1, _NCHUNKS // NS)

    def run(hp_hbm):
        def body(k, carry):
            e0 = pl.multiple_of((s + k * NS) * _ECHUNK, _ECHUNK)
            pltpu.sync_copy(row_hbm.at[pl.ds(e0, _ECHUNK)], row_v)
            pltpu.sync_copy(col_hbm.at[pl.ds(e0, _ECHUNK)], col_v)
            pltpu.async_copy(hp_hbm.at[row_v], rows_v, sem).wait()
            pltpu.sync_copy(rows_v, acc_sh.at[col_v], add=True)
            return carry
        lax.fori_loop(0, nch, body, 0)

    @pl.when(c == 0)
    def _():
        run(hp0_hbm)

    @pl.when(c == 1)
    def _():
        run(hp1_hbm)

    plsc.subcore_barrier()
    pltpu.sync_copy(acc_sh.at[pl.ds(r0, _ROWS_PER_TILE)],
                    out_hbm.at[pl.ds(c * NP + r0, _ROWS_PER_TILE)])


_BN_FIN = 400


def _fin_body(acc_ref, hp0_ref, hp1_ref, wmuI_ref, out_ref):
    rr = lax.broadcasted_iota(jnp.int32, (H, H), 0)
    cc = lax.broadcasted_iota(jnp.int32, (H, H), 1)
    eye = (rr == cc).astype(jnp.float32)
    outs = []
    for t in range(T):
        hp = hp0_ref[...] if t == 0 else hp1_ref[...]
        sacc = acc_ref[t] + hp
        nrm = jnp.sqrt(jnp.sum(sacc * sacc, axis=1, keepdims=True))
        zf = sacc / jnp.maximum(nrm, 1e-12)
        wt = wmuI_ref[t]
        css = jnp.sum(wt * wt, axis=0)
        wtt = lax.dot_general(wt, eye, (((0,), (0,)), ((), ())),
                              preferred_element_type=jnp.float32)
        zi = wtt * (0.8 / jnp.maximum(jnp.sqrt(css), 1e-12))[:, None]
        outs.append(jnp.concatenate([zf, zi], axis=-1))
    out_ref[...] = jnp.stack(outs, axis=0)


def _finalize(acc, hp0, hp1, wmuI):
    return pl.pallas_call(
        _fin_body,
        grid=(N // _BN_FIN,),
        in_specs=[
            pl.BlockSpec((T, _BN_FIN, H), lambda i: (0, i, 0)),
            pl.BlockSpec((_BN_FIN, H), lambda i: (i, 0)),
            pl.BlockSpec((_BN_FIN, H), lambda i: (i, 0)),
            pl.BlockSpec((T, H, _BN_FIN), lambda i: (0, 0, i)),
        ],
        out_specs=pl.BlockSpec((T, _BN_FIN, 2 * H), lambda i: (0, i, 0)),
        out_shape=jax.ShapeDtypeStruct((T, N, 2 * H), jnp.float32),
    )(acc, hp0, hp1, wmuI)


def kernel(x, edge_index, WmuF, bmuF, WlogF, blogF, WmuI, WlogI):
    row = edge_index[0]
    col = edge_index[1]
    wcat = jnp.concatenate([WmuF[0].T, WmuF[1].T], axis=1)
    bcat = jnp.concatenate([bmuF[0], bmuF[1]])[None, :]

    ones_e = jnp.ones((_ECHUNK,), jnp.float32)
    zeros_r = jnp.zeros((_ROWS_PER_TILE,), jnp.float32)
    zeros_rows = jnp.zeros((_ROWS_PER_TILE, H), jnp.float32)

    degp = _deg_kernel(col, ones_e, zeros_r).reshape(NC, NP)
    hp0, hp1 = _matmul(x, wcat, bcat, degp)
    acc = _agg_kernel(row, col, hp0, hp1, zeros_rows).reshape(NC, NP, H)
    return _finalize(acc, hp0, hp1, WmuI)


# speedup vs baseline: 15.5931x; 15.5931x over previous
import functools

import jax
import jax.numpy as jnp
from jax import lax
from jax.experimental import pallas as pl
from jax.experimental.pallas import tpu as pltpu
from jax.experimental.pallas import tpu_sc as plsc

N = 10000
C = 128
H = 128
T = 2
E = 320000

NC = 2
NS = 16
NP = 10240

_ROWS_PER_TILE = NP // NS
_ECHUNK = 128
_NCHUNKS = E // _ECHUNK

_mesh = plsc.VectorSubcoreMesh(core_axis_name="c", subcore_axis_name="s")


@functools.partial(
    pl.kernel,
    out_type=jax.ShapeDtypeStruct((NC * NP,), jnp.float32),
    mesh=_mesh,
    scratch_types=[
        pltpu.VMEM((_ECHUNK,), jnp.int32),
        pltpu.VMEM((_ECHUNK,), jnp.float32),
        pltpu.VMEM_SHARED((NP,), jnp.float32),
    ],
)
def _deg_kernel(col_hbm, ones_hbm, zeros_hbm, out_hbm, col_v, ones_v, deg_sh):
    c = lax.axis_index("c")
    s = lax.axis_index("s")
    wid = c * NS + s
    r0 = s * _ROWS_PER_TILE
    pltpu.sync_copy(zeros_hbm, deg_sh.at[pl.ds(r0, _ROWS_PER_TILE)])
    pltpu.sync_copy(ones_hbm, ones_v)
    plsc.subcore_barrier()

    nch = jnp.where(wid < _NCHUNKS % (NC * NS),
                    _NCHUNKS // (NC * NS) + 1, _NCHUNKS // (NC * NS))

    def body(k, carry):
        e0 = pl.multiple_of((wid + k * NC * NS) * _ECHUNK, _ECHUNK)
        pltpu.sync_copy(col_hbm.at[pl.ds(e0, _ECHUNK)], col_v)
        pltpu.sync_copy(ones_v, deg_sh.at[col_v], add=True)
        return carry

    lax.fori_loop(0, nch, body, 0)
    plsc.subcore_barrier()
    pltpu.sync_copy(deg_sh.at[pl.ds(r0, _ROWS_PER_TILE)],
                    out_hbm.at[pl.ds(c * NP + r0, _ROWS_PER_TILE)])


_BN_MM = 1280


def _mm_body(x_ref, w_ref, b_ref, degp_ref, hp0_ref, hp1_ref):
    dsum = degp_ref[0, :] + degp_ref[1, :] + 1.0
    dinv = lax.rsqrt(dsum)
    h = jnp.dot(x_ref[...], w_ref[...], preferred_element_type=jnp.float32)
    hp = (h + b_ref[...]) * dinv[:, None]
    hp0_ref[...] = hp[:, :H]
    hp1_ref[...] = hp[:, H:]


def _matmul(x, wcat, bcat, degp):
    return pl.pallas_call(
        _mm_body,
        grid=(NP // _BN_MM,),
        in_specs=[
            pl.BlockSpec((_BN_MM, C), lambda i: (i, 0)),
            pl.BlockSpec((C, 2 * H), lambda i: (0, 0)),
            pl.BlockSpec((1, 2 * H), lambda i: (0, 0)),
            pl.BlockSpec((NC, _BN_MM), lambda i: (0, i)),
        ],
        out_specs=[
            pl.BlockSpec((_BN_MM, H), lambda i: (i, 0)),
            pl.BlockSpec((_BN_MM, H), lambda i: (i, 0)),
        ],
        out_shape=[
            jax.ShapeDtypeStruct((N, H), jnp.float32),
            jax.ShapeDtypeStruct((N, H), jnp.float32),
        ],
    )(x, wcat, bcat, degp)


@functools.partial(
    pl.kernel,
    out_type=jax.ShapeDtypeStruct((NC * NP, H), jnp.float32),
    mesh=_mesh,
    scratch_types=[
        pltpu.VMEM((_ECHUNK,), jnp.int32),
        pltpu.VMEM((_ECHUNK,), jnp.int32),
        pltpu.VMEM((_ECHUNK, H), jnp.float32),
        pltpu.VMEM_SHARED((NP, H), jnp.float32),
        pltpu.SemaphoreType.DMA,
    ],
)
def _agg_kernel(row_hbm, col_hbm, hp0_hbm, hp1_hbm, zeros_hbm, out_hbm,
                row_v, col_v, rows_v, acc_sh, sem):
    c = lax.axis_index("c")
    s = lax.axis_index("s")
    r0 = s * _ROWS_PER_TILE
    pltpu.sync_copy(zeros_hbm, acc_sh.at[pl.ds(r0, _ROWS_PER_TILE)])
    plsc.subcore_barrier()

    nch = jnp.where(s < _NCHUNKS % NS, _NCHUNKS // NS + 1, _NCHUNKS // NS)

    def run(hp_hbm):
        def body(k, carry):
            e0 = pl.multiple_of((s + k * NS) * _ECHUNK, _ECHUNK)
            pltpu.sync_copy(row_hbm.at[pl.ds(e0, _ECHUNK)], row_v)
            pltpu.sync_copy(col_hbm.at[pl.ds(e0, _ECHUNK)], col_v)
            pltpu.async_copy(hp_hbm.at[row_v], rows_v, sem).wait()
            pltpu.sync_copy(rows_v, acc_sh.at[col_v], add=True)
            return carry
        lax.fori_loop(0, nch, body, 0)

    @pl.when(c == 0)
    def _():
        run(hp0_hbm)

    @pl.when(c == 1)
    def _():
        run(hp1_hbm)

    plsc.subcore_barrier()
    pltpu.sync_copy(acc_sh.at[pl.ds(r0, _ROWS_PER_TILE)],
                    out_hbm.at[pl.ds(c * NP + r0, _ROWS_PER_TILE)])


_BN_FIN = 512


def _fin_body(acc_ref, hp0_ref, hp1_ref, wmuI_ref, out_ref):
    rr = lax.broadcasted_iota(jnp.int32, (H, H), 0)
    cc = lax.broadcasted_iota(jnp.int32, (H, H), 1)
    eye = (rr == cc).astype(jnp.float32)
    outs = []
    for t in range(T):
        hp = hp0_ref[...] if t == 0 else hp1_ref[...]
        sacc = acc_ref[t] + hp
        nrm = jnp.sqrt(jnp.sum(sacc * sacc, axis=1, keepdims=True))
        zf = sacc / jnp.maximum(nrm, 1e-12)
        wt = wmuI_ref[t]
        css = jnp.sum(wt * wt, axis=0)
        wtt = lax.dot_general(wt, eye, (((0,), (0,)), ((), ())),
                              preferred_element_type=jnp.float32)
        zi = wtt * (0.8 / jnp.maximum(jnp.sqrt(css), 1e-12))[:, None]
        outs.append(jnp.concatenate([zf, zi], axis=-1))
    out_ref[...] = jnp.stack(outs, axis=0)


def _finalize(acc, hp0, hp1, wmuI):
    return pl.pallas_call(
        _fin_body,
        grid=(pl.cdiv(N, _BN_FIN),),
        in_specs=[
            pl.BlockSpec((T, _BN_FIN, H), lambda i: (0, i, 0)),
            pl.BlockSpec((_BN_FIN, H), lambda i: (i, 0)),
            pl.BlockSpec((_BN_FIN, H), lambda i: (i, 0)),
            pl.BlockSpec((T, H, _BN_FIN), lambda i: (0, 0, i)),
        ],
        out_specs=pl.BlockSpec((T, _BN_FIN, 2 * H), lambda i: (0, i, 0)),
        out_shape=jax.ShapeDtypeStruct((T, N, 2 * H), jnp.float32),
    )(acc, hp0, hp1, wmuI)


def kernel(x, edge_index, WmuF, bmuF, WlogF, blogF, WmuI, WlogI):
    row = edge_index[0]
    col = edge_index[1]
    wcat = jnp.concatenate([WmuF[0].T, WmuF[1].T], axis=1)
    bcat = jnp.concatenate([bmuF[0], bmuF[1]])[None, :]

    ones_e = jnp.ones((_ECHUNK,), jnp.float32)
    zeros_r = jnp.zeros((_ROWS_PER_TILE,), jnp.float32)
    zeros_rows = jnp.zeros((_ROWS_PER_TILE, H), jnp.float32)

    degp = _deg_kernel(col, ones_e, zeros_r).reshape(NC, NP)
    hp0, hp1 = _matmul(x, wcat, bcat, degp)
    acc = _agg_kernel(row, col, hp0, hp1, zeros_rows).reshape(NC, NP, H)
    return _finalize(acc, hp0, hp1, WmuI)
